# Initial kernel scaffold; baseline (speedup 1.0000x reference)
import jax
import jax.numpy as jnp
from jax.experimental import pallas as pl


def kernel(x, edge_index, W1, att_src1, att_dst1, b1, W2, att_src2, att_dst2, b2):
    raise NotImplementedError("write your pallas kernel here")


# speedup vs baseline: 45.3049x; 45.3049x over previous
import functools

import jax
import jax.numpy as jnp
from jax import lax
from jax.experimental import pallas as pl
from jax.experimental.pallas import tpu as pltpu
from jax.experimental.pallas import tpu_sc as plsc

_THRESHOLD = -0.1
_EPS_W = 1e-10
_IN_CH = 128
_HID = 16
_OUT_CH = 64
_HEADS = 8
_N = 10000
_E = 320000
_N_PAD = 10240
_BLK = 512
_CHUNK = 128

_NC = 2
_NS = 16
_LANES = 16


def _stage1_body(x_ref, w1_ref, as_ref, ad_ref, g_ref, t1_ref, t2_ref):
    f32 = jnp.float32
    h = jnp.dot(x_ref[...], w1_ref[...], preferred_element_type=f32)
    hp = jax.lax.Precision.HIGHEST
    asg = jnp.dot(h * as_ref[...], g_ref[...], precision=hp,
                  preferred_element_type=f32)
    t2_ref[...] = jnp.dot(h * ad_ref[...], g_ref[...], precision=hp,
                          preferred_element_type=f32)
    t1_ref[...] = jnp.concatenate([h, asg], axis=1)


def _tables_call(x, w1, asf, adf, g):
    n = x.shape[0]
    grid = n // _BLK
    return pl.pallas_call(
        _stage1_body,
        grid=(grid,),
        in_specs=[
            pl.BlockSpec((_BLK, x.shape[1]), lambda i: (i, 0)),
            pl.BlockSpec(w1.shape, lambda i: (0, 0)),
            pl.BlockSpec(asf.shape, lambda i: (0, 0)),
            pl.BlockSpec(adf.shape, lambda i: (0, 0)),
            pl.BlockSpec(g.shape, lambda i: (0, 0)),
        ],
        out_specs=[
            pl.BlockSpec((_BLK, 144), lambda i: (i, 0)),
            pl.BlockSpec((_BLK, 16), lambda i: (i, 0)),
        ],
        out_shape=[
            jax.ShapeDtypeStruct((n, 144), jnp.float32),
            jax.ShapeDtypeStruct((n, 16), jnp.float32),
        ],
    )(x, w1, asf, adf, g)


def _mid_body(p_ref, r1_ref, b1_ref, w2_ref, as2_ref, ad2_ref, g2_ref,
              t1b_ref, t2b_ref):
    f32 = jnp.float32
    s = p_ref[0] + p_ref[1]
    hp = jax.lax.Precision.HIGHEST
    den = jnp.dot(s[:, _IN_CH:_IN_CH + 16], r1_ref[...], precision=hp,
                  preferred_element_type=f32) + 1e-16
    h1 = s[:, 0:_IN_CH] / den + b1_ref[...]
    e1 = jnp.where(h1 > 0.0, h1, jnp.exp(h1) - 1.0)
    h2 = jnp.dot(e1, w2_ref[...], preferred_element_type=f32)
    a2g = jnp.dot(h2 * as2_ref[...], g2_ref[...], precision=hp,
                  preferred_element_type=f32)
    t2b_ref[...] = jnp.dot(h2 * ad2_ref[...], g2_ref[...], precision=hp,
                           preferred_element_type=f32)
    t1b_ref[...] = jnp.concatenate([h2, a2g], axis=1)


def _mid_call(part, r1, b1, w2, as2, ad2, g2):
    n = part.shape[1]
    grid = n // _BLK
    return pl.pallas_call(
        _mid_body,
        grid=(grid,),
        in_specs=[
            pl.BlockSpec((2, _BLK, part.shape[2]), lambda i: (0, i, 0)),
            pl.BlockSpec(r1.shape, lambda i: (0, 0)),
            pl.BlockSpec(b1.shape, lambda i: (0, 0)),
            pl.BlockSpec(w2.shape, lambda i: (0, 0)),
            pl.BlockSpec(as2.shape, lambda i: (0, 0)),
            pl.BlockSpec(ad2.shape, lambda i: (0, 0)),
            pl.BlockSpec(g2.shape, lambda i: (0, 0)),
        ],
        out_specs=[
            pl.BlockSpec((_BLK, 80), lambda i: (i, 0)),
            pl.BlockSpec((_BLK, 16), lambda i: (i, 0)),
        ],
        out_shape=[
            jax.ShapeDtypeStruct((n, 80), jnp.float32),
            jax.ShapeDtypeStruct((n, 16), jnp.float32),
        ],
    )(part, r1, b1, w2, as2, ad2, g2)


def _fin_body(p_ref, s2_ref, b2_ref, o_ref):
    s = p_ref[0] + p_ref[1]
    den = jnp.dot(s[:, _OUT_CH:_OUT_CH + 16], s2_ref[...],
                  precision=jax.lax.Precision.HIGHEST,
                  preferred_element_type=jnp.float32) + 1e-16
    o_ref[...] = s[:, 0:_OUT_CH] / den + b2_ref[...]


def _fin_call(part, s2, b2):
    n = part.shape[1]
    grid = n // _BLK
    return pl.pallas_call(
        _fin_body,
        grid=(grid,),
        in_specs=[
            pl.BlockSpec((2, _BLK, part.shape[2]), lambda i: (0, i, 0)),
            pl.BlockSpec(s2.shape, lambda i: (0, 0)),
            pl.BlockSpec(b2.shape, lambda i: (0, 0)),
        ],
        out_specs=pl.BlockSpec((_BLK, _OUT_CH), lambda i: (i, 0)),
        out_shape=jax.ShapeDtypeStruct((n, _OUT_CH), jnp.float32),
    )(part, s2, b2)


def _make_edge_pass(width, hoff, group_head):
    mesh = plsc.VectorSubcoreMesh(core_axis_name="c", subcore_axis_name="s")
    n_chunks = _E // _CHUNK
    per_core = n_chunks // _NC
    n_iter = (per_core + _NS - 1) // _NS
    rows_per_tile = _N_PAD // _NS

    @functools.partial(
        pl.kernel,
        mesh=mesh,
        compiler_params=pltpu.CompilerParams(use_tc_tiling_on_sc=False),
        out_type=jax.ShapeDtypeStruct((_NC, _N_PAD, width), jnp.float32),
        scratch_types=[
            pltpu.VMEM((_CHUNK,), jnp.int32),
            pltpu.VMEM((_CHUNK,), jnp.int32),
            pltpu.VMEM((_CHUNK, width), jnp.float32),
            pltpu.VMEM((_CHUNK, 16), jnp.float32),
            pltpu.VMEM_SHARED((_N_PAD, width), jnp.float32),
            pltpu.SemaphoreType.DMA,
            pltpu.SemaphoreType.DMA,
        ],
    )
    def edge_pass(t1_hbm, t2_hbm, src_hbm, dst_hbm, zeros_hbm, out_hbm,
                  src_v, dst_v, rows_v, drows_v, acc, sem1, sem2):
        cid = lax.axis_index("c")
        sid = lax.axis_index("s")

        @pl.when(sid == 0)
        def _init():
            pltpu.sync_copy(zeros_hbm, acc)

        plsc.subcore_barrier()

        def chunk_body(i, carry):
            rel = sid + i * _NS

            @pl.when(rel < per_core)
            def _run():
                j = cid * per_core + rel
                base = j * _CHUNK
                pltpu.sync_copy(src_hbm.at[pl.ds(base, _CHUNK)], src_v)
                pltpu.sync_copy(dst_hbm.at[pl.ds(base, _CHUNK)], dst_v)
                pltpu.async_copy(t1_hbm.at[src_v], rows_v, sem1).wait()
                pltpu.async_copy(t2_hbm.at[dst_v], drows_v, sem2).wait()

                def edge_body(e, c2):
                    a = rows_v[e, pl.ds(hoff, 16)] + drows_v[e]
                    lr = jnp.where(a >= 0.0, a, 0.2 * a)
                    w = jnp.where(lr < _THRESHOLD, _EPS_W, jnp.exp(lr))
                    dnums = lax.GatherDimensionNumbers(
                        offset_dims=(), collapsed_slice_dims=(0,),
                        start_index_map=(0,))
                    for g, h in enumerate(group_head):
                        wh = lax.gather(
                            w, jnp.full((_LANES, 1), h, jnp.int32), dnums,
                            slice_sizes=(1,),
                            mode=lax.GatherScatterMode.PROMISE_IN_BOUNDS)
                        seg = rows_v[e, pl.ds(g * 16, 16)]
                        rows_v[e, pl.ds(g * 16, 16)] = seg * wh
                    rows_v[e, pl.ds(hoff, 16)] = w
                    return c2

                lax.fori_loop(0, _CHUNK, edge_body, 0)
                pltpu.sync_copy(rows_v, acc.at[dst_v], add=True)

            return carry

        lax.fori_loop(0, n_iter, chunk_body, 0)
        plsc.subcore_barrier()
        r0 = sid * rows_per_tile
        pltpu.sync_copy(acc.at[pl.ds(r0, rows_per_tile)],
                        out_hbm.at[cid, pl.ds(r0, rows_per_tile)])

    return edge_pass


_edge_pass_1 = _make_edge_pass(144, 128, tuple(range(8)))
_edge_pass_2 = _make_edge_pass(80, 64, (0, 0, 0, 0))


def kernel(x, edge_index, W1, att_src1, att_dst1, b1, W2, att_src2, att_dst2,
           b2):
    f32 = jnp.float32
    src = edge_index[0].astype(jnp.int32)
    dst = edge_index[1].astype(jnp.int32)
    x_pad = jnp.pad(x.astype(f32), ((0, _N_PAD - _N), (0, 0)))

    as1f = att_src1.reshape(1, _IN_CH).astype(f32)
    ad1f = att_dst1.reshape(1, _IN_CH).astype(f32)
    as2f = att_src2.reshape(1, _OUT_CH).astype(f32)
    ad2f = att_dst2.reshape(1, _OUT_CH).astype(f32)
    g1 = jnp.concatenate(
        [jnp.repeat(jnp.eye(_HEADS, dtype=f32), _HID, axis=0),
         jnp.zeros((_IN_CH, 8), f32)], axis=1)
    g2 = jnp.concatenate(
        [jnp.ones((_OUT_CH, 1), f32),
         jnp.zeros((_OUT_CH, 15), f32)], axis=1)

    r1 = jnp.concatenate(
        [jnp.repeat(jnp.eye(_HEADS, dtype=f32), _HID, axis=1),
         jnp.zeros((8, _IN_CH), f32)], axis=0)
    s2 = jnp.concatenate(
        [jnp.ones((1, _OUT_CH), f32),
         jnp.zeros((15, _OUT_CH), f32)], axis=0)

    b1r = b1.reshape(1, _IN_CH).astype(f32)
    b2r = b2.reshape(1, _OUT_CH).astype(f32)
    z144 = jnp.zeros((_N_PAD, 144), f32)
    z80 = jnp.zeros((_N_PAD, 80), f32)

    t1, t2 = _tables_call(x_pad, W1.astype(f32), as1f, ad1f, g1)
    part1 = _edge_pass_1(t1, t2, src, dst, z144)
    t1b, t2b = _mid_call(part1, r1, b1r, W2.astype(f32), as2f, ad2f, g2)
    part2 = _edge_pass_2(t1b, t2b, src, dst, z80)
    out = _fin_call(part2, s2, b2r)
    return out[:_N]


# speedup vs baseline: 53.5551x; 1.1821x over previous
import functools

import jax
import jax.numpy as jnp
from jax import lax
from jax.experimental import pallas as pl
from jax.experimental.pallas import tpu as pltpu
from jax.experimental.pallas import tpu_sc as plsc

_THRESHOLD = -0.1
_EPS_W = 1e-10
_IN_CH = 128
_HID = 16
_OUT_CH = 64
_HEADS = 8
_N = 10000
_E = 320000
_N_PAD = 10240
_BLK = 512
_CHUNK = 128

_NC = 2
_NS = 16
_LANES = 16


def _stage1_body(x_ref, w1_ref, as_ref, ad_ref, g_ref, t1_ref, t2_ref):
    f32 = jnp.float32
    h = jnp.dot(x_ref[...], w1_ref[...], preferred_element_type=f32)
    hp = jax.lax.Precision.HIGHEST
    asg = jnp.dot(h * as_ref[...], g_ref[...], precision=hp,
                  preferred_element_type=f32)
    t2_ref[...] = jnp.dot(h * ad_ref[...], g_ref[...], precision=hp,
                          preferred_element_type=f32)
    t1_ref[...] = jnp.concatenate([h, asg], axis=1)


def _tables_call(x, w1, asf, adf, g):
    n = x.shape[0]
    grid = n // _BLK
    return pl.pallas_call(
        _stage1_body,
        grid=(grid,),
        in_specs=[
            pl.BlockSpec((_BLK, x.shape[1]), lambda i: (i, 0)),
            pl.BlockSpec(w1.shape, lambda i: (0, 0)),
            pl.BlockSpec(asf.shape, lambda i: (0, 0)),
            pl.BlockSpec(adf.shape, lambda i: (0, 0)),
            pl.BlockSpec(g.shape, lambda i: (0, 0)),
        ],
        out_specs=[
            pl.BlockSpec((_BLK, 144), lambda i: (i, 0)),
            pl.BlockSpec((_BLK, 16), lambda i: (i, 0)),
        ],
        out_shape=[
            jax.ShapeDtypeStruct((n, 144), jnp.float32),
            jax.ShapeDtypeStruct((n, 16), jnp.float32),
        ],
    )(x, w1, asf, adf, g)


def _mid_body(p_ref, r1_ref, b1_ref, w2_ref, as2_ref, ad2_ref, g2_ref,
              t1b_ref, t2b_ref):
    f32 = jnp.float32
    s = p_ref[0] + p_ref[1]
    hp = jax.lax.Precision.HIGHEST
    den = jnp.dot(s[:, _IN_CH:_IN_CH + 16], r1_ref[...], precision=hp,
                  preferred_element_type=f32) + 1e-16
    h1 = s[:, 0:_IN_CH] / den + b1_ref[...]
    e1 = jnp.where(h1 > 0.0, h1, jnp.exp(h1) - 1.0)
    h2 = jnp.dot(e1, w2_ref[...], preferred_element_type=f32)
    a2g = jnp.dot(h2 * as2_ref[...], g2_ref[...], precision=hp,
                  preferred_element_type=f32)
    t2b_ref[...] = jnp.dot(h2 * ad2_ref[...], g2_ref[...], precision=hp,
                           preferred_element_type=f32)
    t1b_ref[...] = jnp.concatenate([h2, a2g], axis=1)


def _mid_call(part, r1, b1, w2, as2, ad2, g2):
    n = part.shape[1]
    grid = n // _BLK
    return pl.pallas_call(
        _mid_body,
        grid=(grid,),
        in_specs=[
            pl.BlockSpec((2, _BLK, part.shape[2]), lambda i: (0, i, 0)),
            pl.BlockSpec(r1.shape, lambda i: (0, 0)),
            pl.BlockSpec(b1.shape, lambda i: (0, 0)),
            pl.BlockSpec(w2.shape, lambda i: (0, 0)),
            pl.BlockSpec(as2.shape, lambda i: (0, 0)),
            pl.BlockSpec(ad2.shape, lambda i: (0, 0)),
            pl.BlockSpec(g2.shape, lambda i: (0, 0)),
        ],
        out_specs=[
            pl.BlockSpec((_BLK, 80), lambda i: (i, 0)),
            pl.BlockSpec((_BLK, 16), lambda i: (i, 0)),
        ],
        out_shape=[
            jax.ShapeDtypeStruct((n, 80), jnp.float32),
            jax.ShapeDtypeStruct((n, 16), jnp.float32),
        ],
    )(part, r1, b1, w2, as2, ad2, g2)


def _fin_body(p_ref, s2_ref, b2_ref, o_ref):
    s = p_ref[0] + p_ref[1]
    den = jnp.dot(s[:, _OUT_CH:_OUT_CH + 16], s2_ref[...],
                  precision=jax.lax.Precision.HIGHEST,
                  preferred_element_type=jnp.float32) + 1e-16
    o_ref[...] = s[:, 0:_OUT_CH] / den + b2_ref[...]


def _fin_call(part, s2, b2):
    n = part.shape[1]
    grid = n // _BLK
    return pl.pallas_call(
        _fin_body,
        grid=(grid,),
        in_specs=[
            pl.BlockSpec((2, _BLK, part.shape[2]), lambda i: (0, i, 0)),
            pl.BlockSpec(s2.shape, lambda i: (0, 0)),
            pl.BlockSpec(b2.shape, lambda i: (0, 0)),
        ],
        out_specs=pl.BlockSpec((_BLK, _OUT_CH), lambda i: (i, 0)),
        out_shape=jax.ShapeDtypeStruct((n, _OUT_CH), jnp.float32),
    )(part, s2, b2)


def _make_edge_pass(width, hoff, group_head):
    mesh = plsc.VectorSubcoreMesh(core_axis_name="c", subcore_axis_name="s")
    n_chunks = _E // _CHUNK
    per_core = n_chunks // _NC
    n_iter = (per_core + _NS - 1) // _NS
    rows_per_tile = _N_PAD // _NS

    @functools.partial(
        pl.kernel,
        mesh=mesh,
        compiler_params=pltpu.CompilerParams(use_tc_tiling_on_sc=False),
        out_type=jax.ShapeDtypeStruct((_NC, _N_PAD, width), jnp.float32),
        scratch_types=[
            pltpu.VMEM((_CHUNK,), jnp.int32),
            pltpu.VMEM((_CHUNK,), jnp.int32),
            pltpu.VMEM((_CHUNK, width), jnp.float32),
            pltpu.VMEM((_CHUNK, 16), jnp.float32),
            pltpu.VMEM_SHARED((_N_PAD, width), jnp.float32),
            pltpu.SemaphoreType.DMA,
            pltpu.SemaphoreType.DMA,
        ],
    )
    def edge_pass(t1_hbm, t2_hbm, src_hbm, dst_hbm, zeros_hbm, out_hbm,
                  src_v, dst_v, rows_v, drows_v, acc, sem1, sem2):
        cid = lax.axis_index("c")
        sid = lax.axis_index("s")

        @pl.when(sid == 0)
        def _init():
            pltpu.sync_copy(zeros_hbm, acc)

        plsc.subcore_barrier()

        def chunk_body(i, carry):
            rel = sid + i * _NS

            @pl.when(rel < per_core)
            def _run():
                j = cid * per_core + rel
                base = j * _CHUNK
                pltpu.sync_copy(src_hbm.at[pl.ds(base, _CHUNK)], src_v)
                pltpu.sync_copy(dst_hbm.at[pl.ds(base, _CHUNK)], dst_v)
                pltpu.async_copy(t1_hbm.at[src_v], rows_v, sem1).wait()
                pltpu.async_copy(t2_hbm.at[dst_v], drows_v, sem2).wait()

                @plsc.parallel_loop(0, _CHUNK, unroll=8)
                def edge_body(e):
                    a = rows_v[e, pl.ds(hoff, 16)] + drows_v[e]
                    lr = jnp.where(a >= 0.0, a, 0.2 * a)
                    w = jnp.where(lr < _THRESHOLD, _EPS_W, jnp.exp(lr))
                    dnums = lax.GatherDimensionNumbers(
                        offset_dims=(), collapsed_slice_dims=(0,),
                        start_index_map=(0,))
                    for g, h in enumerate(group_head):
                        wh = lax.gather(
                            w, jnp.full((_LANES, 1), h, jnp.int32), dnums,
                            slice_sizes=(1,),
                            mode=lax.GatherScatterMode.PROMISE_IN_BOUNDS)
                        seg = rows_v[e, pl.ds(g * 16, 16)]
                        rows_v[e, pl.ds(g * 16, 16)] = seg * wh
                    rows_v[e, pl.ds(hoff, 16)] = w
                pltpu.sync_copy(rows_v, acc.at[dst_v], add=True)

            return carry

        lax.fori_loop(0, n_iter, chunk_body, 0)
        plsc.subcore_barrier()
        r0 = sid * rows_per_tile
        pltpu.sync_copy(acc.at[pl.ds(r0, rows_per_tile)],
                        out_hbm.at[cid, pl.ds(r0, rows_per_tile)])

    return edge_pass


_edge_pass_1 = _make_edge_pass(144, 128, tuple(range(8)))
_edge_pass_2 = _make_edge_pass(80, 64, (0, 0, 0, 0))


def kernel(x, edge_index, W1, att_src1, att_dst1, b1, W2, att_src2, att_dst2,
           b2):
    f32 = jnp.float32
    src = edge_index[0].astype(jnp.int32)
    dst = edge_index[1].astype(jnp.int32)
    x_pad = jnp.pad(x.astype(f32), ((0, _N_PAD - _N), (0, 0)))

    as1f = att_src1.reshape(1, _IN_CH).astype(f32)
    ad1f = att_dst1.reshape(1, _IN_CH).astype(f32)
    as2f = att_src2.reshape(1, _OUT_CH).astype(f32)
    ad2f = att_dst2.reshape(1, _OUT_CH).astype(f32)
    g1 = jnp.concatenate(
        [jnp.repeat(jnp.eye(_HEADS, dtype=f32), _HID, axis=0),
         jnp.zeros((_IN_CH, 8), f32)], axis=1)
    g2 = jnp.concatenate(
        [jnp.ones((_OUT_CH, 1), f32),
         jnp.zeros((_OUT_CH, 15), f32)], axis=1)

    r1 = jnp.concatenate(
        [jnp.repeat(jnp.eye(_HEADS, dtype=f32), _HID, axis=1),
         jnp.zeros((8, _IN_CH), f32)], axis=0)
    s2 = jnp.concatenate(
        [jnp.ones((1, _OUT_CH), f32),
         jnp.zeros((15, _OUT_CH), f32)], axis=0)

    b1r = b1.reshape(1, _IN_CH).astype(f32)
    b2r = b2.reshape(1, _OUT_CH).astype(f32)
    z144 = jnp.zeros((_N_PAD, 144), f32)
    z80 = jnp.zeros((_N_PAD, 80), f32)

    t1, t2 = _tables_call(x_pad, W1.astype(f32), as1f, ad1f, g1)
    part1 = _edge_pass_1(t1, t2, src, dst, z144)
    t1b, t2b = _mid_call(part1, r1, b1r, W2.astype(f32), as2f, ad2f, g2)
    part2 = _edge_pass_2(t1b, t2b, src, dst, z80)
    out = _fin_call(part2, s2, b2r)
    return out[:_N]


# speedup vs baseline: 83.0220x; 1.5502x over previous
import functools

import jax
import jax.numpy as jnp
from jax import lax
from jax.experimental import pallas as pl
from jax.experimental.pallas import tpu as pltpu
from jax.experimental.pallas import tpu_sc as plsc

_THRESHOLD = -0.1
_EPS_W = 1e-10
_IN_CH = 128
_HID = 16
_OUT_CH = 64
_HEADS = 8
_N = 10000
_E = 320000
_N_PAD = 10112
_BLK = 632
_CHUNK = 80

_NC = 2
_NS = 16
_LANES = 16


def _stage1_body(x_ref, w1_ref, as_ref, ad_ref, g_ref, t1_ref, t2_ref):
    f32 = jnp.float32
    h = jnp.dot(x_ref[...], w1_ref[...], preferred_element_type=f32)
    hp = jax.lax.Precision.HIGHEST
    asg = jnp.dot(h * as_ref[...], g_ref[...], precision=hp,
                  preferred_element_type=f32)
    t2_ref[...] = jnp.dot(h * ad_ref[...], g_ref[...], precision=hp,
                          preferred_element_type=f32)
    t1_ref[...] = jnp.concatenate([h, asg], axis=1)


def _tables_call(x, w1, asf, adf, g):
    n = x.shape[0]
    grid = n // _BLK
    return pl.pallas_call(
        _stage1_body,
        grid=(grid,),
        in_specs=[
            pl.BlockSpec((_BLK, x.shape[1]), lambda i: (i, 0)),
            pl.BlockSpec(w1.shape, lambda i: (0, 0)),
            pl.BlockSpec(asf.shape, lambda i: (0, 0)),
            pl.BlockSpec(adf.shape, lambda i: (0, 0)),
            pl.BlockSpec(g.shape, lambda i: (0, 0)),
        ],
        out_specs=[
            pl.BlockSpec((_BLK, 144), lambda i: (i, 0)),
            pl.BlockSpec((_BLK, 16), lambda i: (i, 0)),
        ],
        out_shape=[
            jax.ShapeDtypeStruct((n, 144), jnp.float32),
            jax.ShapeDtypeStruct((n, 16), jnp.float32),
        ],
    )(x, w1, asf, adf, g)


def _mid_body(p_ref, r1_ref, b1_ref, w2_ref, as2_ref, ad2_ref, g2_ref,
              t1b_ref, t2b_ref):
    f32 = jnp.float32
    s = p_ref[0] + p_ref[1]
    hp = jax.lax.Precision.HIGHEST
    den = jnp.dot(s[:, _IN_CH:_IN_CH + 16], r1_ref[...], precision=hp,
                  preferred_element_type=f32) + 1e-16
    h1 = s[:, 0:_IN_CH] / den + b1_ref[...]
    e1 = jnp.where(h1 > 0.0, h1, jnp.exp(h1) - 1.0)
    h2 = jnp.dot(e1, w2_ref[...], preferred_element_type=f32)
    a2g = jnp.dot(h2 * as2_ref[...], g2_ref[...], precision=hp,
                  preferred_element_type=f32)
    t2b_ref[...] = jnp.dot(h2 * ad2_ref[...], g2_ref[...], precision=hp,
                           preferred_element_type=f32)
    t1b_ref[...] = jnp.concatenate([h2, a2g], axis=1)


def _mid_call(part, r1, b1, w2, as2, ad2, g2):
    n = part.shape[1]
    grid = n // _BLK
    return pl.pallas_call(
        _mid_body,
        grid=(grid,),
        in_specs=[
            pl.BlockSpec((2, _BLK, part.shape[2]), lambda i: (0, i, 0)),
            pl.BlockSpec(r1.shape, lambda i: (0, 0)),
            pl.BlockSpec(b1.shape, lambda i: (0, 0)),
            pl.BlockSpec(w2.shape, lambda i: (0, 0)),
            pl.BlockSpec(as2.shape, lambda i: (0, 0)),
            pl.BlockSpec(ad2.shape, lambda i: (0, 0)),
            pl.BlockSpec(g2.shape, lambda i: (0, 0)),
        ],
        out_specs=[
            pl.BlockSpec((_BLK, 80), lambda i: (i, 0)),
            pl.BlockSpec((_BLK, 16), lambda i: (i, 0)),
        ],
        out_shape=[
            jax.ShapeDtypeStruct((n, 80), jnp.float32),
            jax.ShapeDtypeStruct((n, 16), jnp.float32),
        ],
    )(part, r1, b1, w2, as2, ad2, g2)


def _fin_body(p_ref, s2_ref, b2_ref, o_ref):
    s = p_ref[0] + p_ref[1]
    den = jnp.dot(s[:, _OUT_CH:_OUT_CH + 16], s2_ref[...],
                  precision=jax.lax.Precision.HIGHEST,
                  preferred_element_type=jnp.float32) + 1e-16
    o_ref[...] = s[:, 0:_OUT_CH] / den + b2_ref[...]


def _fin_call(part, s2, b2):
    n = part.shape[1]
    grid = n // _BLK
    return pl.pallas_call(
        _fin_body,
        grid=(grid,),
        in_specs=[
            pl.BlockSpec((2, _BLK, part.shape[2]), lambda i: (0, i, 0)),
            pl.BlockSpec(s2.shape, lambda i: (0, 0)),
            pl.BlockSpec(b2.shape, lambda i: (0, 0)),
        ],
        out_specs=pl.BlockSpec((_BLK, _OUT_CH), lambda i: (i, 0)),
        out_shape=jax.ShapeDtypeStruct((n, _OUT_CH), jnp.float32),
    )(part, s2, b2)


def _make_edge_pass(width, hoff, group_head):
    mesh = plsc.VectorSubcoreMesh(core_axis_name="c", subcore_axis_name="s")
    n_chunks = _E // _CHUNK
    per_core = n_chunks // _NC
    n_iter = (per_core + _NS - 1) // _NS
    rows_per_tile = _N_PAD // _NS

    n_groups = (n_iter + 1) // 2

    @functools.partial(
        pl.kernel,
        mesh=mesh,
        compiler_params=pltpu.CompilerParams(use_tc_tiling_on_sc=False),
        out_type=jax.ShapeDtypeStruct((_NC, _N_PAD, width), jnp.float32),
        scratch_types=[
            pltpu.VMEM((2, _CHUNK), jnp.int32),
            pltpu.VMEM((2, _CHUNK), jnp.int32),
            pltpu.VMEM((_CHUNK, width), jnp.float32),
            pltpu.VMEM((_CHUNK, width), jnp.float32),
            pltpu.VMEM((_CHUNK, 16), jnp.float32),
            pltpu.VMEM((_CHUNK, 16), jnp.float32),
            pltpu.VMEM_SHARED((_N_PAD, width), jnp.float32),
            pltpu.SemaphoreType.DMA,
            pltpu.SemaphoreType.DMA,
            pltpu.SemaphoreType.DMA,
            pltpu.SemaphoreType.DMA,
        ],
    )
    def edge_pass(t1_hbm, t2_hbm, ei_hbm, zeros_hbm, out_hbm,
                  iv0, iv1, rows_v0, rows_v1,
                  drows_v0, drows_v1, acc, sg1_0, sg1_1, sg2_0, sg2_1):
        cid = lax.axis_index("c")
        sid = lax.axis_index("s")
        bufs = ((iv0, rows_v0, drows_v0, sg1_0, sg2_0),
                (iv1, rows_v1, drows_v1, sg1_1, sg2_1))

        @pl.when(sid == 0)
        def _init():
            pltpu.sync_copy(zeros_hbm, acc)

        plsc.subcore_barrier()

        def issue(k, b):
            iv, rv, drv, s1, s2 = bufs[b]

            @pl.when(sid + k * _NS < per_core)
            def _():
                base = (cid * per_core + sid + k * _NS) * _CHUNK
                pltpu.sync_copy(ei_hbm.at[:, pl.ds(base, _CHUNK)], iv)
                pltpu.async_copy(t1_hbm.at[iv.at[0]], rv, s1)
                pltpu.async_copy(t2_hbm.at[iv.at[1]], drv, s2)

        def process(k, b):
            iv, rv, drv, s1, s2 = bufs[b]

            @pl.when(sid + k * _NS < per_core)
            def _compute():
                pltpu.make_async_copy(t1_hbm.at[iv.at[0]], rv, s1).wait()
                pltpu.make_async_copy(t2_hbm.at[iv.at[1]], drv, s2).wait()
                @plsc.parallel_loop(0, _CHUNK, unroll=8)
                def edge_body(e):
                    a = rv[e, pl.ds(hoff, 16)] + drv[e]
                    lr = jnp.where(a >= 0.0, a, 0.2 * a)
                    w = jnp.where(lr < _THRESHOLD, _EPS_W, jnp.exp(lr))
                    dnums = lax.GatherDimensionNumbers(
                        offset_dims=(), collapsed_slice_dims=(0,),
                        start_index_map=(0,))
                    for g, h in enumerate(group_head):
                        wh = lax.gather(
                            w, jnp.full((_LANES, 1), h, jnp.int32), dnums,
                            slice_sizes=(1,),
                            mode=lax.GatherScatterMode.PROMISE_IN_BOUNDS)
                        seg = rv[e, pl.ds(g * 16, 16)]
                        rv[e, pl.ds(g * 16, 16)] = seg * wh
                    rv[e, pl.ds(hoff, 16)] = w
                pltpu.sync_copy(rv, acc.at[iv.at[1]], add=True)

            issue(k + 2, b)

        issue(0, 0)
        issue(1, 1)

        def group_body(i, carry):
            process(2 * i, 0)
            process(2 * i + 1, 1)
            return carry

        lax.fori_loop(0, n_groups, group_body, 0)
        plsc.subcore_barrier()
        r0 = sid * rows_per_tile
        pltpu.sync_copy(acc.at[pl.ds(r0, rows_per_tile)],
                        out_hbm.at[cid, pl.ds(r0, rows_per_tile)])

    return edge_pass


_edge_pass_1 = _make_edge_pass(144, 128, tuple(range(8)))
_edge_pass_2 = _make_edge_pass(80, 64, (0, 0, 0, 0))


def kernel(x, edge_index, W1, att_src1, att_dst1, b1, W2, att_src2, att_dst2,
           b2):
    f32 = jnp.float32
    ei32 = edge_index.astype(jnp.int32)
    x_pad = jnp.pad(x.astype(f32), ((0, _N_PAD - _N), (0, 0)))

    as1f = att_src1.reshape(1, _IN_CH).astype(f32)
    ad1f = att_dst1.reshape(1, _IN_CH).astype(f32)
    as2f = att_src2.reshape(1, _OUT_CH).astype(f32)
    ad2f = att_dst2.reshape(1, _OUT_CH).astype(f32)
    g1 = jnp.concatenate(
        [jnp.repeat(jnp.eye(_HEADS, dtype=f32), _HID, axis=0),
         jnp.zeros((_IN_CH, 8), f32)], axis=1)
    g2 = jnp.concatenate(
        [jnp.ones((_OUT_CH, 1), f32),
         jnp.zeros((_OUT_CH, 15), f32)], axis=1)

    r1 = jnp.concatenate(
        [jnp.repeat(jnp.eye(_HEADS, dtype=f32), _HID, axis=1),
         jnp.zeros((8, _IN_CH), f32)], axis=0)
    s2 = jnp.concatenate(
        [jnp.ones((1, _OUT_CH), f32),
         jnp.zeros((15, _OUT_CH), f32)], axis=0)

    b1r = b1.reshape(1, _IN_CH).astype(f32)
    b2r = b2.reshape(1, _OUT_CH).astype(f32)
    z144 = jnp.zeros((_N_PAD, 144), f32)
    z80 = jnp.zeros((_N_PAD, 80), f32)

    t1, t2 = _tables_call(x_pad, W1.astype(f32), as1f, ad1f, g1)
    part1 = _edge_pass_1(t1, t2, ei32, z144)
    t1b, t2b = _mid_call(part1, r1, b1r, W2.astype(f32), as2f, ad2f, g2)
    part2 = _edge_pass_2(t1b, t2b, ei32, z80)
    out = _fin_call(part2, s2, b2r)
    return out[:_N]


# speedup vs baseline: 85.9623x; 1.0354x over previous
import functools

import jax
import jax.numpy as jnp
from jax import lax
from jax.experimental import pallas as pl
from jax.experimental.pallas import tpu as pltpu
from jax.experimental.pallas import tpu_sc as plsc

_THRESHOLD = -0.1
_EPS_W = 1e-10
_IN_CH = 128
_HID = 16
_OUT_CH = 64
_HEADS = 8
_N = 10000
_E = 320000
_N_PAD = 10112
_BLK = 632
_CHUNK = 80

_NC = 2
_NS = 16
_LANES = 16


def _stage1_body(x_ref, w1_ref, as_ref, ad_ref, g_ref, t1_ref, t2_ref):
    f32 = jnp.float32
    h = jnp.dot(x_ref[...], w1_ref[...], preferred_element_type=f32)
    hp = jax.lax.Precision.HIGHEST
    asg = jnp.dot(h * as_ref[...], g_ref[...], precision=hp,
                  preferred_element_type=f32)
    t2_ref[...] = jnp.dot(h * ad_ref[...], g_ref[...], precision=hp,
                          preferred_element_type=f32)
    t1_ref[...] = jnp.concatenate([h, asg], axis=1)


def _tables_call(x, w1, asf, adf, g):
    n = x.shape[0]
    grid = n // _BLK
    return pl.pallas_call(
        _stage1_body,
        grid=(grid,),
        in_specs=[
            pl.BlockSpec((_BLK, x.shape[1]), lambda i: (i, 0)),
            pl.BlockSpec(w1.shape, lambda i: (0, 0)),
            pl.BlockSpec(asf.shape, lambda i: (0, 0)),
            pl.BlockSpec(adf.shape, lambda i: (0, 0)),
            pl.BlockSpec(g.shape, lambda i: (0, 0)),
        ],
        out_specs=[
            pl.BlockSpec((_BLK, 144), lambda i: (i, 0)),
            pl.BlockSpec((_BLK, 16), lambda i: (i, 0)),
        ],
        out_shape=[
            jax.ShapeDtypeStruct((n, 144), jnp.float32),
            jax.ShapeDtypeStruct((n, 16), jnp.float32),
        ],
    )(x, w1, asf, adf, g)


def _mid_body(p_ref, r1_ref, b1_ref, w2_ref, as2_ref, ad2_ref, g2_ref,
              t1b_ref, t2b_ref):
    f32 = jnp.float32
    s = p_ref[0] + p_ref[1]
    hp = jax.lax.Precision.HIGHEST
    den = jnp.dot(s[:, _IN_CH:_IN_CH + 16], r1_ref[...], precision=hp,
                  preferred_element_type=f32) + 1e-16
    h1 = s[:, 0:_IN_CH] / den + b1_ref[...]
    e1 = jnp.where(h1 > 0.0, h1, jnp.exp(h1) - 1.0)
    h2 = jnp.dot(e1, w2_ref[...], preferred_element_type=f32)
    a2g = jnp.dot(h2 * as2_ref[...], g2_ref[...], precision=hp,
                  preferred_element_type=f32)
    t2b_ref[...] = jnp.dot(h2 * ad2_ref[...], g2_ref[...], precision=hp,
                           preferred_element_type=f32)
    t1b_ref[...] = jnp.concatenate([h2, a2g], axis=1)


def _mid_call(part, r1, b1, w2, as2, ad2, g2):
    n = part.shape[1]
    grid = n // _BLK
    return pl.pallas_call(
        _mid_body,
        grid=(grid,),
        in_specs=[
            pl.BlockSpec((2, _BLK, part.shape[2]), lambda i: (0, i, 0)),
            pl.BlockSpec(r1.shape, lambda i: (0, 0)),
            pl.BlockSpec(b1.shape, lambda i: (0, 0)),
            pl.BlockSpec(w2.shape, lambda i: (0, 0)),
            pl.BlockSpec(as2.shape, lambda i: (0, 0)),
            pl.BlockSpec(ad2.shape, lambda i: (0, 0)),
            pl.BlockSpec(g2.shape, lambda i: (0, 0)),
        ],
        out_specs=[
            pl.BlockSpec((_BLK, 80), lambda i: (i, 0)),
            pl.BlockSpec((_BLK, 16), lambda i: (i, 0)),
        ],
        out_shape=[
            jax.ShapeDtypeStruct((n, 80), jnp.float32),
            jax.ShapeDtypeStruct((n, 16), jnp.float32),
        ],
    )(part, r1, b1, w2, as2, ad2, g2)


def _fin_body(p_ref, s2_ref, b2_ref, o_ref):
    s = p_ref[0] + p_ref[1]
    den = jnp.dot(s[:, _OUT_CH:_OUT_CH + 16], s2_ref[...],
                  precision=jax.lax.Precision.HIGHEST,
                  preferred_element_type=jnp.float32) + 1e-16
    o_ref[...] = s[:, 0:_OUT_CH] / den + b2_ref[...]


def _fin_call(part, s2, b2):
    n = part.shape[1]
    grid = n // _BLK
    return pl.pallas_call(
        _fin_body,
        grid=(grid,),
        in_specs=[
            pl.BlockSpec((2, _BLK, part.shape[2]), lambda i: (0, i, 0)),
            pl.BlockSpec(s2.shape, lambda i: (0, 0)),
            pl.BlockSpec(b2.shape, lambda i: (0, 0)),
        ],
        out_specs=pl.BlockSpec((_BLK, _OUT_CH), lambda i: (i, 0)),
        out_shape=jax.ShapeDtypeStruct((n, _OUT_CH), jnp.float32),
    )(part, s2, b2)


def _make_edge_pass(width, hoff, group_head, chunk):
    mesh = plsc.VectorSubcoreMesh(core_axis_name="c", subcore_axis_name="s")
    _CHUNK = chunk
    n_chunks = _E // _CHUNK
    per_core = n_chunks // _NC
    n_iter = (per_core + _NS - 1) // _NS
    rows_per_tile = _N_PAD // _NS

    n_groups = (n_iter + 1) // 2

    @functools.partial(
        pl.kernel,
        mesh=mesh,
        compiler_params=pltpu.CompilerParams(use_tc_tiling_on_sc=False),
        out_type=jax.ShapeDtypeStruct((_NC, _N_PAD, width), jnp.float32),
        scratch_types=[
            pltpu.VMEM((2, _CHUNK), jnp.int32),
            pltpu.VMEM((2, _CHUNK), jnp.int32),
            pltpu.VMEM((_CHUNK, width), jnp.float32),
            pltpu.VMEM((_CHUNK, width), jnp.float32),
            pltpu.VMEM((_CHUNK, 16), jnp.float32),
            pltpu.VMEM((_CHUNK, 16), jnp.float32),
            pltpu.VMEM_SHARED((_N_PAD, width), jnp.float32),
            pltpu.SemaphoreType.DMA,
            pltpu.SemaphoreType.DMA,
            pltpu.SemaphoreType.DMA,
            pltpu.SemaphoreType.DMA,
        ],
    )
    def edge_pass(t1_hbm, t2_hbm, ei_hbm, zeros_hbm, out_hbm,
                  iv0, iv1, rows_v0, rows_v1,
                  drows_v0, drows_v1, acc, sg1_0, sg1_1, sg2_0, sg2_1):
        cid = lax.axis_index("c")
        sid = lax.axis_index("s")
        bufs = ((iv0, rows_v0, drows_v0, sg1_0, sg2_0),
                (iv1, rows_v1, drows_v1, sg1_1, sg2_1))

        @pl.when(sid == 0)
        def _init():
            pltpu.sync_copy(zeros_hbm, acc)

        plsc.subcore_barrier()

        def issue(k, b):
            iv, rv, drv, s1, s2 = bufs[b]

            @pl.when(sid + k * _NS < per_core)
            def _():
                base = (cid * per_core + sid + k * _NS) * _CHUNK
                pltpu.sync_copy(ei_hbm.at[:, pl.ds(base, _CHUNK)], iv)
                pltpu.async_copy(t1_hbm.at[iv.at[0]], rv, s1)
                pltpu.async_copy(t2_hbm.at[iv.at[1]], drv, s2)

        def process(k, b):
            iv, rv, drv, s1, s2 = bufs[b]

            @pl.when(sid + k * _NS < per_core)
            def _compute():
                pltpu.make_async_copy(t1_hbm.at[iv.at[0]], rv, s1).wait()
                pltpu.make_async_copy(t2_hbm.at[iv.at[1]], drv, s2).wait()
                @plsc.parallel_loop(0, _CHUNK, unroll=8)
                def edge_body(e):
                    a = rv[e, pl.ds(hoff, 16)] + drv[e]
                    lr = jnp.where(a >= 0.0, a, 0.2 * a)
                    w = jnp.where(lr < _THRESHOLD, _EPS_W, jnp.exp(lr))
                    dnums = lax.GatherDimensionNumbers(
                        offset_dims=(), collapsed_slice_dims=(0,),
                        start_index_map=(0,))
                    for g, h in enumerate(group_head):
                        wh = lax.gather(
                            w, jnp.full((_LANES, 1), h, jnp.int32), dnums,
                            slice_sizes=(1,),
                            mode=lax.GatherScatterMode.PROMISE_IN_BOUNDS)
                        seg = rv[e, pl.ds(g * 16, 16)]
                        rv[e, pl.ds(g * 16, 16)] = seg * wh
                    rv[e, pl.ds(hoff, 16)] = w
                pltpu.sync_copy(rv, acc.at[iv.at[1]], add=True)

            issue(k + 2, b)

        issue(0, 0)
        issue(1, 1)

        def group_body(i, carry):
            process(2 * i, 0)
            process(2 * i + 1, 1)
            return carry

        lax.fori_loop(0, n_groups, group_body, 0)
        plsc.subcore_barrier()
        r0 = sid * rows_per_tile
        pltpu.sync_copy(acc.at[pl.ds(r0, rows_per_tile)],
                        out_hbm.at[cid, pl.ds(r0, rows_per_tile)])

    return edge_pass


_edge_pass_1 = _make_edge_pass(144, 128, tuple(range(8)), 80)
_edge_pass_2 = _make_edge_pass(80, 64, (0, 0, 0, 0), 128)


def kernel(x, edge_index, W1, att_src1, att_dst1, b1, W2, att_src2, att_dst2,
           b2):
    f32 = jnp.float32
    ei32 = edge_index.astype(jnp.int32)
    x_pad = jnp.pad(x.astype(f32), ((0, _N_PAD - _N), (0, 0)))

    as1f = att_src1.reshape(1, _IN_CH).astype(f32)
    ad1f = att_dst1.reshape(1, _IN_CH).astype(f32)
    as2f = att_src2.reshape(1, _OUT_CH).astype(f32)
    ad2f = att_dst2.reshape(1, _OUT_CH).astype(f32)
    g1 = jnp.concatenate(
        [jnp.repeat(jnp.eye(_HEADS, dtype=f32), _HID, axis=0),
         jnp.zeros((_IN_CH, 8), f32)], axis=1)
    g2 = jnp.concatenate(
        [jnp.ones((_OUT_CH, 1), f32),
         jnp.zeros((_OUT_CH, 15), f32)], axis=1)

    r1 = jnp.concatenate(
        [jnp.repeat(jnp.eye(_HEADS, dtype=f32), _HID, axis=1),
         jnp.zeros((8, _IN_CH), f32)], axis=0)
    s2 = jnp.concatenate(
        [jnp.ones((1, _OUT_CH), f32),
         jnp.zeros((15, _OUT_CH), f32)], axis=0)

    b1r = b1.reshape(1, _IN_CH).astype(f32)
    b2r = b2.reshape(1, _OUT_CH).astype(f32)
    z144 = jnp.zeros((_N_PAD, 144), f32)
    z80 = jnp.zeros((_N_PAD, 80), f32)

    t1, t2 = _tables_call(x_pad, W1.astype(f32), as1f, ad1f, g1)
    part1 = _edge_pass_1(t1, t2, ei32, z144)
    t1b, t2b = _mid_call(part1, r1, b1r, W2.astype(f32), as2f, ad2f, g2)
    part2 = _edge_pass_2(t1b, t2b, ei32, z80)
    out = _fin_call(part2, s2, b2r)
    return out[:_N]


# speedup vs baseline: 103.3534x; 1.2023x over previous
import functools

import jax
import jax.numpy as jnp
from jax import lax
from jax.experimental import pallas as pl
from jax.experimental.pallas import tpu as pltpu
from jax.experimental.pallas import tpu_sc as plsc

_THRESHOLD = -0.1
_EPS_W = 1e-10
_IN_CH = 128
_HID = 16
_OUT_CH = 64
_HEADS = 8
_N = 10000
_E = 320000
_N_PAD = 10112
_BLK = 632
_CHUNK = 80

_NC = 2
_NS = 16
_LANES = 16


def _stage1_body(x_ref, w1_ref, as_ref, ad_ref, g_ref, t1_ref, t2_ref):
    f32 = jnp.float32
    h = jnp.dot(x_ref[...], w1_ref[...], preferred_element_type=f32)
    hp = jax.lax.Precision.HIGHEST
    asg = jnp.dot(h * as_ref[...], g_ref[...], precision=hp,
                  preferred_element_type=f32)
    t2_ref[...] = jnp.dot(h * ad_ref[...], g_ref[...], precision=hp,
                          preferred_element_type=f32)
    t1_ref[...] = jnp.concatenate([h, asg], axis=1)


def _tables_call(x, w1, asf, adf, g):
    n = x.shape[0]
    grid = n // _BLK
    return pl.pallas_call(
        _stage1_body,
        grid=(grid,),
        in_specs=[
            pl.BlockSpec((_BLK, x.shape[1]), lambda i: (i, 0)),
            pl.BlockSpec(w1.shape, lambda i: (0, 0)),
            pl.BlockSpec(asf.shape, lambda i: (0, 0)),
            pl.BlockSpec(adf.shape, lambda i: (0, 0)),
            pl.BlockSpec(g.shape, lambda i: (0, 0)),
        ],
        out_specs=[
            pl.BlockSpec((_BLK, 144), lambda i: (i, 0)),
            pl.BlockSpec((_BLK, 16), lambda i: (i, 0)),
        ],
        out_shape=[
            jax.ShapeDtypeStruct((n, 144), jnp.float32),
            jax.ShapeDtypeStruct((n, 16), jnp.float32),
        ],
    )(x, w1, asf, adf, g)


def _mid_body(p_ref, r1_ref, b1_ref, w2_ref, as2_ref, ad2_ref, g2_ref,
              t1b_ref, t2b_ref):
    f32 = jnp.float32
    s = p_ref[0] + p_ref[1]
    hp = jax.lax.Precision.HIGHEST
    den = jnp.dot(s[:, _IN_CH:_IN_CH + 16], r1_ref[...], precision=hp,
                  preferred_element_type=f32) + 1e-16
    h1 = s[:, 0:_IN_CH] / den + b1_ref[...]
    e1 = jnp.where(h1 > 0.0, h1, jnp.exp(h1) - 1.0)
    h2 = jnp.dot(e1, w2_ref[...], preferred_element_type=f32)
    a2g = jnp.dot(h2 * as2_ref[...], g2_ref[...], precision=hp,
                  preferred_element_type=f32)
    t2b_ref[...] = jnp.dot(h2 * ad2_ref[...], g2_ref[...], precision=hp,
                           preferred_element_type=f32)
    t1b_ref[...] = jnp.concatenate([h2, a2g], axis=1)


def _mid_call(part, r1, b1, w2, as2, ad2, g2):
    n = part.shape[1]
    grid = n // _BLK
    return pl.pallas_call(
        _mid_body,
        grid=(grid,),
        in_specs=[
            pl.BlockSpec((2, _BLK, part.shape[2]), lambda i: (0, i, 0)),
            pl.BlockSpec(r1.shape, lambda i: (0, 0)),
            pl.BlockSpec(b1.shape, lambda i: (0, 0)),
            pl.BlockSpec(w2.shape, lambda i: (0, 0)),
            pl.BlockSpec(as2.shape, lambda i: (0, 0)),
            pl.BlockSpec(ad2.shape, lambda i: (0, 0)),
            pl.BlockSpec(g2.shape, lambda i: (0, 0)),
        ],
        out_specs=[
            pl.BlockSpec((_BLK, 80), lambda i: (i, 0)),
            pl.BlockSpec((_BLK, 16), lambda i: (i, 0)),
        ],
        out_shape=[
            jax.ShapeDtypeStruct((n, 80), jnp.float32),
            jax.ShapeDtypeStruct((n, 16), jnp.float32),
        ],
    )(part, r1, b1, w2, as2, ad2, g2)


def _fin_body(p_ref, s2_ref, b2_ref, o_ref):
    s = p_ref[0] + p_ref[1]
    den = jnp.dot(s[:, _OUT_CH:_OUT_CH + 16], s2_ref[...],
                  precision=jax.lax.Precision.HIGHEST,
                  preferred_element_type=jnp.float32) + 1e-16
    o_ref[...] = s[:, 0:_OUT_CH] / den + b2_ref[...]


def _fin_call(part, s2, b2):
    n = part.shape[1]
    grid = n // _BLK
    return pl.pallas_call(
        _fin_body,
        grid=(grid,),
        in_specs=[
            pl.BlockSpec((2, _BLK, part.shape[2]), lambda i: (0, i, 0)),
            pl.BlockSpec(s2.shape, lambda i: (0, 0)),
            pl.BlockSpec(b2.shape, lambda i: (0, 0)),
        ],
        out_specs=pl.BlockSpec((_BLK, _OUT_CH), lambda i: (i, 0)),
        out_shape=jax.ShapeDtypeStruct((n, _OUT_CH), jnp.float32),
    )(part, s2, b2)


def _make_edge_pass(width, hoff, group_head, chunk):
    mesh = plsc.VectorSubcoreMesh(core_axis_name="c", subcore_axis_name="s")
    _CHUNK = chunk
    n_chunks = _E // _CHUNK
    per_core = n_chunks // _NC
    n_iter = (per_core + _NS - 1) // _NS
    rows_per_tile = _N_PAD // _NS

    n_groups = (n_iter + 1) // 2

    @functools.partial(
        pl.kernel,
        mesh=mesh,
        compiler_params=pltpu.CompilerParams(use_tc_tiling_on_sc=False),
        out_type=jax.ShapeDtypeStruct((_NC, _N_PAD, width), jnp.float32),
        scratch_types=[
            pltpu.VMEM((2, _CHUNK), jnp.int32),
            pltpu.VMEM((2, _CHUNK), jnp.int32),
            pltpu.VMEM((_CHUNK, width), jnp.float32),
            pltpu.VMEM((_CHUNK, width), jnp.float32),
            pltpu.VMEM((_CHUNK, 16), jnp.float32),
            pltpu.VMEM((_CHUNK, 16), jnp.float32),
            pltpu.VMEM_SHARED((_N_PAD, width), jnp.float32),
            pltpu.SemaphoreType.DMA,
            pltpu.SemaphoreType.DMA,
            pltpu.SemaphoreType.DMA,
            pltpu.SemaphoreType.DMA,
        ],
    )
    def edge_pass(t1_hbm, t2_hbm, ei_hbm, zeros_hbm, out_hbm,
                  iv0, iv1, rows_v0, rows_v1,
                  drows_v0, drows_v1, acc, sg1_0, sg1_1, sg2_0, sg2_1):
        cid = lax.axis_index("c")
        sid = lax.axis_index("s")
        bufs = ((iv0, rows_v0, drows_v0, sg1_0, sg2_0),
                (iv1, rows_v1, drows_v1, sg1_1, sg2_1))

        @pl.when(sid == 0)
        def _init():
            pltpu.sync_copy(zeros_hbm, acc)

        plsc.subcore_barrier()

        def issue(k, b):
            iv, rv, drv, s1, s2 = bufs[b]

            @pl.when(sid + k * _NS < per_core)
            def _():
                base = (cid * per_core + sid + k * _NS) * _CHUNK
                pltpu.sync_copy(ei_hbm.at[:, pl.ds(base, _CHUNK)], iv)
                pltpu.async_copy(t1_hbm.at[iv.at[0]], rv, s1)
                pltpu.async_copy(t2_hbm.at[iv.at[1]], drv, s2)

        def process(k, b):
            iv, rv, drv, s1, s2 = bufs[b]

            @pl.when(sid + k * _NS < per_core)
            def _compute():
                pltpu.make_async_copy(t1_hbm.at[iv.at[0]], rv, s1).wait()
                pltpu.make_async_copy(t2_hbm.at[iv.at[1]], drv, s2).wait()
                @plsc.parallel_loop(0, _CHUNK, unroll=16)
                def edge_body(e):
                    a = rv[e, pl.ds(hoff, 16)] + drv[e]
                    lr = jnp.where(a >= 0.0, a, 0.2 * a)
                    w = jnp.where(lr < _THRESHOLD, _EPS_W, jnp.exp(lr))
                    dnums = lax.GatherDimensionNumbers(
                        offset_dims=(), collapsed_slice_dims=(0,),
                        start_index_map=(0,))
                    for g, h in enumerate(group_head):
                        wh = lax.gather(
                            w, jnp.full((_LANES, 1), h, jnp.int32), dnums,
                            slice_sizes=(1,),
                            mode=lax.GatherScatterMode.PROMISE_IN_BOUNDS)
                        seg = rv[e, pl.ds(g * 16, 16)]
                        rv[e, pl.ds(g * 16, 16)] = seg * wh
                    rv[e, pl.ds(hoff, 16)] = w
                pltpu.sync_copy(rv, acc.at[iv.at[1]], add=True)

            issue(k + 2, b)

        issue(0, 0)
        issue(1, 1)

        def group_body(i, carry):
            process(2 * i, 0)
            process(2 * i + 1, 1)
            return carry

        lax.fori_loop(0, n_groups, group_body, 0)
        plsc.subcore_barrier()
        r0 = sid * rows_per_tile
        pltpu.sync_copy(acc.at[pl.ds(r0, rows_per_tile)],
                        out_hbm.at[cid, pl.ds(r0, rows_per_tile)])

    return edge_pass


_edge_pass_1 = _make_edge_pass(144, 128, tuple(range(8)), 80)
_edge_pass_2 = _make_edge_pass(80, 64, (0, 0, 0, 0), 128)


def kernel(x, edge_index, W1, att_src1, att_dst1, b1, W2, att_src2, att_dst2,
           b2):
    f32 = jnp.float32
    ei32 = edge_index.astype(jnp.int32)
    x_pad = jnp.pad(x.astype(f32), ((0, _N_PAD - _N), (0, 0)))

    as1f = att_src1.reshape(1, _IN_CH).astype(f32)
    ad1f = att_dst1.reshape(1, _IN_CH).astype(f32)
    as2f = att_src2.reshape(1, _OUT_CH).astype(f32)
    ad2f = att_dst2.reshape(1, _OUT_CH).astype(f32)
    g1 = jnp.concatenate(
        [jnp.repeat(jnp.eye(_HEADS, dtype=f32), _HID, axis=0),
         jnp.zeros((_IN_CH, 8), f32)], axis=1)
    g2 = jnp.concatenate(
        [jnp.ones((_OUT_CH, 1), f32),
         jnp.zeros((_OUT_CH, 15), f32)], axis=1)

    r1 = jnp.concatenate(
        [jnp.repeat(jnp.eye(_HEADS, dtype=f32), _HID, axis=1),
         jnp.zeros((8, _IN_CH), f32)], axis=0)
    s2 = jnp.concatenate(
        [jnp.ones((1, _OUT_CH), f32),
         jnp.zeros((15, _OUT_CH), f32)], axis=0)

    b1r = b1.reshape(1, _IN_CH).astype(f32)
    b2r = b2.reshape(1, _OUT_CH).astype(f32)
    z144 = jnp.zeros((_N_PAD, 144), f32)
    z80 = jnp.zeros((_N_PAD, 80), f32)

    t1, t2 = _tables_call(x_pad, W1.astype(f32), as1f, ad1f, g1)
    part1 = _edge_pass_1(t1, t2, ei32, z144)
    t1b, t2b = _mid_call(part1, r1, b1r, W2.astype(f32), as2f, ad2f, g2)
    part2 = _edge_pass_2(t1b, t2b, ei32, z80)
    out = _fin_call(part2, s2, b2r)
    return out[:_N]
